# baseline (speedup 1.0000x reference)
import functools

import jax
import jax.numpy as jnp
from jax import lax
from jax.experimental import pallas as pl
from jax.experimental.pallas import tpu as pltpu
from jax.experimental.pallas import tpu_sc as plsc

BATCH = 16384
FACTOR = 16
MLP_DIM = 64
LINE = 128

_NC = 2
_NS = 16
_NW = _NC * _NS
_BPW = BATCH // _NW
_CHUNK = 128
_NCHUNK = _BPW // _CHUNK


def _sc_gather(u8, i8, u2, i2, t_u16, t_i16, t_u64, t_i64):
    mesh = plsc.VectorSubcoreMesh(core_axis_name="c", subcore_axis_name="s")
    out_line = jax.ShapeDtypeStruct((BATCH, LINE), jnp.float32)

    @functools.partial(
        pl.kernel,
        out_type=[out_line, out_line, out_line, out_line],
        mesh=mesh,
        scratch_types=[
            pltpu.VMEM((_BPW,), jnp.int32),
            pltpu.VMEM((_BPW,), jnp.int32),
            pltpu.VMEM((_BPW,), jnp.int32),
            pltpu.VMEM((_BPW,), jnp.int32),
            pltpu.VMEM((_CHUNK, LINE), jnp.float32),
            pltpu.VMEM((_CHUNK, LINE), jnp.float32),
            pltpu.VMEM((_CHUNK, LINE), jnp.float32),
            pltpu.VMEM((_CHUNK, LINE), jnp.float32),
            pltpu.SemaphoreType.DMA,
        ],
    )
    def k(u8_h, i8_h, u2_h, i2_h, tu16_h, ti16_h, tu64_h, ti64_h,
          ou16_h, oi16_h, ou64_h, oi64_h,
          u8v, i8v, u2v, i2v, bu16, bi16, bu64, bi64, sem):
        wid = lax.axis_index("s") * _NC + lax.axis_index("c")
        base = wid * _BPW
        pltpu.sync_copy(u8_h.at[pl.ds(base, _BPW)], u8v)
        pltpu.sync_copy(i8_h.at[pl.ds(base, _BPW)], i8v)
        pltpu.sync_copy(u2_h.at[pl.ds(base, _BPW)], u2v)
        pltpu.sync_copy(i2_h.at[pl.ds(base, _BPW)], i2v)
        for c in range(_NCHUNK):
            sl = pl.ds(c * _CHUNK, _CHUNK)
            copies = [
                pltpu.async_copy(tu16_h.at[u8v.at[sl]], bu16, sem),
                pltpu.async_copy(ti16_h.at[i8v.at[sl]], bi16, sem),
                pltpu.async_copy(tu64_h.at[u2v.at[sl]], bu64, sem),
                pltpu.async_copy(ti64_h.at[i2v.at[sl]], bi64, sem),
            ]
            for cp in copies:
                cp.wait()
            osl = pl.ds(base + c * _CHUNK, _CHUNK)
            pltpu.sync_copy(bu16, ou16_h.at[osl])
            pltpu.sync_copy(bi16, oi16_h.at[osl])
            pltpu.sync_copy(bu64, ou64_h.at[osl])
            pltpu.sync_copy(bi64, oi64_h.at[osl])

    return k(u8, i8, u2, i2, t_u16, t_i16, t_u64, t_i64)


_BB = 2048


def _tc_body(ru16, ri16, ru64, ri64, ou8, oi8, ou2, oi2,
             w0a, w0b, b0, w1, b1, w2, b2, wp, bp, out):
    def pick16(rows, off):
        acc = jnp.zeros((_BB, FACTOR), jnp.float32)
        o = off[...]
        for kk in range(8):
            acc = acc + jnp.where(o == kk, rows[:, kk * FACTOR:(kk + 1) * FACTOR], 0.0)
        return acc

    ug = pick16(ru16[...], ou8)
    ig = pick16(ri16[...], oi8)
    gmf = ug * ig
    hu_rows = ru64[...]
    hi_rows = ri64[...]
    hu = jnp.where(ou2[...] == 0, hu_rows[:, :MLP_DIM], hu_rows[:, MLP_DIM:])
    hi = jnp.where(oi2[...] == 0, hi_rows[:, :MLP_DIM], hi_rows[:, MLP_DIM:])
    h = hu @ w0a[...] + hi @ w0b[...] + b0[...]
    h = jnp.maximum(h, 0.0)
    h = jnp.maximum(h @ w1[...] + b1[...], 0.0)
    h = jnp.maximum(h @ w2[...] + b2[...], 0.0)
    fused = jnp.concatenate([gmf, h], axis=-1)
    out[...] = jnp.sum(fused * wp[...], axis=-1) + bp[0]


def _tc_dense(ru16, ri16, ru64, ri64, ou8, oi8, ou2, oi2,
              W0, b0, W1, b1, W2, b2, Wp, bp):
    grid = (BATCH // _BB,)

    def row_blk(shape):
        return pl.BlockSpec((_BB,) + shape[1:], lambda i: (i,) + (0,) * (len(shape) - 1))

    def full_blk(shape):
        return pl.BlockSpec(shape, lambda i: (0,) * len(shape))

    w0a, w0b = W0[:MLP_DIM], W0[MLP_DIM:]
    b0r, b1r, b2r = b0.reshape(1, -1), b1.reshape(1, -1), b2.reshape(1, -1)
    wpr = Wp.reshape(1, -1)
    in_specs = [
        row_blk((BATCH, LINE)), row_blk((BATCH, LINE)),
        row_blk((BATCH, LINE)), row_blk((BATCH, LINE)),
        row_blk((BATCH, 1)), row_blk((BATCH, 1)),
        row_blk((BATCH, 1)), row_blk((BATCH, 1)),
        full_blk(w0a.shape), full_blk(w0b.shape), full_blk(b0r.shape),
        full_blk(W1.shape), full_blk(b1r.shape),
        full_blk(W2.shape), full_blk(b2r.shape),
        full_blk(wpr.shape), full_blk(bp.shape),
    ]
    return pl.pallas_call(
        _tc_body,
        grid=grid,
        in_specs=in_specs,
        out_specs=pl.BlockSpec((_BB,), lambda i: (i,)),
        out_shape=jax.ShapeDtypeStruct((BATCH,), jnp.float32),
    )(ru16, ri16, ru64, ri64, ou8, oi8, ou2, oi2,
      w0a, w0b, b0r, W1, b1r, W2, b2r, wpr, bp)


def kernel(user, item, user_emb_gmf, item_emb_gmf, user_emb_mlp, item_emb_mlp,
           W0, b0, W1, b1, W2, b2, Wp, bp):
    u = user.astype(jnp.int32)
    it = item.astype(jnp.int32)
    t_u16 = user_emb_gmf.reshape(-1, LINE)
    t_i16 = item_emb_gmf.reshape(-1, LINE)
    t_u64 = user_emb_mlp.reshape(-1, LINE)
    t_i64 = item_emb_mlp.reshape(-1, LINE)
    u8, i8 = u >> 3, it >> 3
    u2, i2 = u >> 1, it >> 1
    ru16, ri16, ru64, ri64 = _sc_gather(u8, i8, u2, i2, t_u16, t_i16, t_u64, t_i64)
    ou8 = (u & 7).reshape(-1, 1)
    oi8 = (it & 7).reshape(-1, 1)
    ou2 = (u & 1).reshape(-1, 1)
    oi2 = (it & 1).reshape(-1, 1)
    return _tc_dense(ru16, ri16, ru64, ri64, ou8, oi8, ou2, oi2,
                     W0, b0, W1, b1, W2, b2, Wp, bp)
